# baseline (speedup 1.0000x reference)
import functools

import jax
import jax.numpy as jnp
from jax import lax
from jax.experimental import pallas as pl
from jax.experimental.pallas import tpu as pltpu
from jax.experimental.pallas import tpu_sc as plsc

P = 512
PAD = 16
W = P + 2 * PAD
B = 32
L = 16
NCHUNK = W // L
NU = P // L
SLABW = 128

_cached = {}


def _build():
    info = plsc.get_sparse_core_info()
    nc = info.num_cores
    mesh = plsc.VectorSubcoreMesh(core_axis_name="c", subcore_axis_name="s")

    scratch = [
        pltpu.VMEM((P + PAD, SLABW), jnp.float32),
        pltpu.VMEM((5 * L, SLABW), jnp.float32),
        pltpu.VMEM((5 * L, SLABW), jnp.float32),
        pltpu.VMEM((L + 1, W), jnp.float32),
        pltpu.VMEM((L + 1, W), jnp.float32),
    ] + [pltpu.SemaphoreType.DMA] * 5

    @functools.partial(
        pl.kernel,
        mesh=mesh,
        out_type=jax.ShapeDtypeStruct((B, P, W), jnp.float32),
        compiler_params=pltpu.CompilerParams(needs_layout_passes=False),
        scratch_types=scratch,
    )
    def shear_kernel(t01, t02, out, S, T0, T1, O0, O1, sS, sT0, sT1, sO0, sO1):
        b = lax.axis_index("s") * nc + lax.axis_index("c")
        iota = lax.iota(jnp.int32, L)
        zf = jnp.zeros((L,), jnp.float32)
        Tb, Ob = (T0, T1), (O0, O1)
        sT, sO = (sT0, sT1), (sO0, sO1)
        for rr in range(P, P + PAD):
            for cc8 in range(SLABW // L):
                S[rr, pl.ds(L * cc8, L)] = zf
        for Tp in Tb:
            for rr in range(4 * L, 5 * L):
                for cc8 in range(SLABW // L):
                    Tp[rr, pl.ds(L * cc8, L)] = zf

        slab_rows = [P, 400, 272, 144]

        def unit_i0(g):
            return (P - SLABW) - SLABW * (g >> 3) + L * (g & 7)

        def issue_slab(j):
            for s in range(4):
                @pl.when(j == 4 * s)
                def _(s=s):
                    nr = slab_rows[s]
                    pltpu.async_copy(
                        t02.at[b, pl.ds(0, nr), pl.ds(SLABW * s, SLABW)],
                        S.at[pl.ds(0, nr)], sS)
                    pltpu.make_async_copy(
                        t02.at[b, pl.ds(0, nr), pl.ds(0, SLABW)],
                        S.at[pl.ds(0, nr)], sS).wait()

        def issue_in(g, p):
            i0 = unit_i0(g)
            for tt in range(4):
                pltpu.async_copy(
                    t01.at[b, pl.ds(i0, L), pl.ds(SLABW * tt, SLABW)],
                    Tb[p].at[pl.ds(L * tt, L)], sT[p])

        def wait_in(p):
            for tt in range(4):
                pltpu.make_async_copy(
                    t01.at[b, pl.ds(0, L), pl.ds(SLABW * tt, SLABW)],
                    Tb[p].at[pl.ds(L * tt, L)], sT[p]).wait()

        def wait_out(p):
            pltpu.make_async_copy(Ob[p].at[pl.ds(0, L)],
                                  out.at[b, pl.ds(0, L), :], sO[p]).wait()

        def compute_unit(g, p):
            i0 = unit_i0(g)
            q = i0 // L
            cslab = (P - 1) - i0 - SLABW * (g >> 3)
            T, O = Tb[p], Ob[p]

            def row_body(li):
                i = i0 + li
                ccv = (cslab - li) + iota * 0
                liv = li + iota * 0

                @plsc.parallel_loop(0, q + 1, unroll=16,
                                    carry=(PAD + i) - iota)
                def _(m, rv):
                    O[li, pl.ds(m * L, L)] = plsc.load_gather(S, [rv, ccv])
                    return rv - L

                rm = liv - iota
                gS = plsc.load_gather(S, [jnp.maximum(rm, 0), ccv])
                gT = plsc.load_gather(T, [liv, jnp.maximum(iota - li - 1, 0)])
                O[li, pl.ds((q + 1) * L, L)] = jnp.where(rm >= 0, gS, gT)

                cv0 = (L - 1 - li - (q + 2) * L) + iota

                @plsc.parallel_loop(q + 2, NCHUNK, unroll=16)
                def _(m):
                    cv = cv0 + m * L
                    rowT = ((cv >> 7) << 4) + liv
                    O[li, pl.ds(m * L, L)] = plsc.load_gather(
                        T, [rowT, cv & (SLABW - 1)])

            plsc.parallel_loop(0, L)(row_body)
            pltpu.async_copy(O.at[pl.ds(0, L)], out.at[b, pl.ds(i0, L), :],
                             sO[p])

        issue_in(0, 0)
        issue_in(1, 1)

        def pair_body(j, carry):
            issue_slab(j)
            for p in (0, 1):
                g = 2 * j + p
                wait_in(p)

                @pl.when(j > 0)
                def _():
                    wait_out(p)

                compute_unit(g, p)

                @pl.when(j < NU // 2 - 1)
                def _():
                    issue_in(g + 2, p)
            return carry

        lax.fori_loop(0, NU // 2, pair_body, 0)
        wait_out(0)
        wait_out(1)

    return shear_kernel


def kernel(triangle01, triangle02):
    if "k" not in _cached:
        _cached["k"] = _build()
    return _cached["k"](triangle01, triangle02)


# speedup vs baseline: 1.1212x; 1.1212x over previous
import functools

import jax
import jax.numpy as jnp
from jax import lax
from jax.experimental import pallas as pl
from jax.experimental.pallas import tpu as pltpu
from jax.experimental.pallas import tpu_sc as plsc

P = 512
PAD = 16
W = P + 2 * PAD
B = 32
L = 16
NCHUNK = W // L
NU = P // L
SLABW = 128

_cached = {}


def _build():
    info = plsc.get_sparse_core_info()
    nc = info.num_cores
    mesh = plsc.VectorSubcoreMesh(core_axis_name="c", subcore_axis_name="s")

    scratch = [
        pltpu.VMEM((P + PAD, SLABW), jnp.float32),
        pltpu.VMEM((5 * L, SLABW), jnp.float32),
        pltpu.VMEM((5 * L, SLABW), jnp.float32),
        pltpu.VMEM((L + 1, W), jnp.float32),
        pltpu.VMEM((L + 1, W), jnp.float32),
    ] + [pltpu.SemaphoreType.DMA] * 5

    @functools.partial(
        pl.kernel,
        mesh=mesh,
        out_type=jax.ShapeDtypeStruct((B, P, W), jnp.float32),
        compiler_params=pltpu.CompilerParams(needs_layout_passes=False),
        scratch_types=scratch,
    )
    def shear_kernel(t01, t02, out, S, T0, T1, O0, O1, sS, sT0, sT1, sO0, sO1):
        b = lax.axis_index("s") * nc + lax.axis_index("c")
        iota = lax.iota(jnp.int32, L)
        zf = jnp.zeros((L,), jnp.float32)
        Tb, Ob = (T0, T1), (O0, O1)
        sT, sO = (sT0, sT1), (sO0, sO1)
        for rr in range(P, P + PAD):
            for cc8 in range(SLABW // L):
                S[rr, pl.ds(L * cc8, L)] = zf
        for Tp in Tb:
            for rr in range(4 * L, 5 * L):
                for cc8 in range(SLABW // L):
                    Tp[rr, pl.ds(L * cc8, L)] = zf

        slab_rows = [P, 400, 272, 144]

        def unit_i0(g):
            return (P - SLABW) - SLABW * (g >> 3) + L * (g & 7)

        def issue_slab(j):
            for s in range(4):
                @pl.when(j == 4 * s)
                def _(s=s):
                    nr = slab_rows[s]
                    pltpu.async_copy(
                        t02.at[b, pl.ds(0, nr), pl.ds(SLABW * s, SLABW)],
                        S.at[pl.ds(0, nr)], sS)
                    pltpu.make_async_copy(
                        t02.at[b, pl.ds(0, nr), pl.ds(0, SLABW)],
                        S.at[pl.ds(0, nr)], sS).wait()

        def issue_in(g, p):
            i0 = unit_i0(g)
            for tt in range(4):
                pltpu.async_copy(
                    t01.at[b, pl.ds(i0, L), pl.ds(SLABW * tt, SLABW)],
                    Tb[p].at[pl.ds(L * tt, L)], sT[p])

        def wait_in(p):
            for tt in range(4):
                pltpu.make_async_copy(
                    t01.at[b, pl.ds(0, L), pl.ds(SLABW * tt, SLABW)],
                    Tb[p].at[pl.ds(L * tt, L)], sT[p]).wait()

        def wait_out(p):
            pltpu.make_async_copy(Ob[p].at[pl.ds(0, L)],
                                  out.at[b, pl.ds(0, L), :], sO[p]).wait()

        def compute_unit(g, p):
            i0 = unit_i0(g)
            q = i0 // L
            cslab = (P - 1) - i0 - SLABW * (g >> 3)
            T, O = Tb[p], Ob[p]

            def row_body(li):
                i = i0 + li
                ccv = (cslab - li) + iota * 0
                liv = li + iota * 0

                @plsc.parallel_loop(0, q + 1, unroll=8,
                                    carry=(PAD + i) - iota)
                def _(m, rv):
                    O[li, pl.ds(m * L, L)] = plsc.load_gather(S, [rv, ccv])
                    return rv - L

                rm = liv - iota
                gS = plsc.load_gather(S, [jnp.maximum(rm, 0), ccv])
                gT = plsc.load_gather(T, [liv, jnp.maximum(iota - li - 1, 0)])
                O[li, pl.ds((q + 1) * L, L)] = jnp.where(rm >= 0, gS, gT)

                cv0 = (L - 1 - li - (q + 2) * L) + iota

                @plsc.parallel_loop(q + 2, NCHUNK, unroll=8)
                def _(m):
                    cv = cv0 + m * L
                    rowT = ((cv >> 7) << 4) + liv
                    O[li, pl.ds(m * L, L)] = plsc.load_gather(
                        T, [rowT, cv & (SLABW - 1)])

            plsc.parallel_loop(0, L)(row_body)
            pltpu.async_copy(O.at[pl.ds(0, L)], out.at[b, pl.ds(i0, L), :],
                             sO[p])

        issue_in(0, 0)
        issue_in(1, 1)

        def pair_body(j, carry):
            issue_slab(j)
            for p in (0, 1):
                g = 2 * j + p
                wait_in(p)

                @pl.when(j > 0)
                def _():
                    wait_out(p)

                compute_unit(g, p)

                @pl.when(j < NU // 2 - 1)
                def _():
                    issue_in(g + 2, p)
            return carry

        lax.fori_loop(0, NU // 2, pair_body, 0)
        wait_out(0)
        wait_out(1)

    return shear_kernel


def kernel(triangle01, triangle02):
    if "k" not in _cached:
        _cached["k"] = _build()
    return _cached["k"](triangle01, triangle02)


# speedup vs baseline: 1.1220x; 1.0007x over previous
import functools

import jax
import jax.numpy as jnp
from jax import lax
from jax.experimental import pallas as pl
from jax.experimental.pallas import tpu as pltpu
from jax.experimental.pallas import tpu_sc as plsc

P = 512
PAD = 16
W = P + 2 * PAD
B = 32
L = 16
NCHUNK = W // L
NU = P // L
SLABW = 128

_cached = {}


def _build():
    info = plsc.get_sparse_core_info()
    nc = info.num_cores
    mesh = plsc.VectorSubcoreMesh(core_axis_name="c", subcore_axis_name="s")

    scratch = [
        pltpu.VMEM((P + PAD, SLABW), jnp.float32),
        pltpu.VMEM((5 * L, SLABW), jnp.float32),
        pltpu.VMEM((5 * L, SLABW), jnp.float32),
        pltpu.VMEM((L + 1, W), jnp.float32),
        pltpu.VMEM((L + 1, W), jnp.float32),
    ] + [pltpu.SemaphoreType.DMA] * 5

    @functools.partial(
        pl.kernel,
        mesh=mesh,
        out_type=jax.ShapeDtypeStruct((B, P, W), jnp.float32),
        compiler_params=pltpu.CompilerParams(needs_layout_passes=False),
        scratch_types=scratch,
    )
    def shear_kernel(t01, t02, out, S, T0, T1, O0, O1, sS, sT0, sT1, sO0, sO1):
        b = lax.axis_index("s") * nc + lax.axis_index("c")
        iota = lax.iota(jnp.int32, L)
        zf = jnp.zeros((L,), jnp.float32)
        Tb, Ob = (T0, T1), (O0, O1)
        sT, sO = (sT0, sT1), (sO0, sO1)
        for rr in range(P, P + PAD):
            for cc8 in range(SLABW // L):
                S[rr, pl.ds(L * cc8, L)] = zf
        for Tp in Tb:
            for rr in range(4 * L, 5 * L):
                for cc8 in range(SLABW // L):
                    Tp[rr, pl.ds(L * cc8, L)] = zf

        slab_rows = [P, 400, 272, 144]

        def unit_i0(g):
            return (P - SLABW) - SLABW * (g >> 3) + L * (g & 7)

        def issue_slab(j):
            for s in range(4):
                @pl.when(j == 4 * s)
                def _(s=s):
                    nr = slab_rows[s]
                    pltpu.async_copy(
                        t02.at[b, pl.ds(0, nr), pl.ds(SLABW * s, SLABW)],
                        S.at[pl.ds(0, nr)], sS)
                    pltpu.make_async_copy(
                        t02.at[b, pl.ds(0, nr), pl.ds(0, SLABW)],
                        S.at[pl.ds(0, nr)], sS).wait()

        def issue_in(g, p):
            i0 = unit_i0(g)
            for tt in range(4):
                pltpu.async_copy(
                    t01.at[b, pl.ds(i0, L), pl.ds(SLABW * tt, SLABW)],
                    Tb[p].at[pl.ds(L * tt, L)], sT[p])

        def wait_in(p):
            for tt in range(4):
                pltpu.make_async_copy(
                    t01.at[b, pl.ds(0, L), pl.ds(SLABW * tt, SLABW)],
                    Tb[p].at[pl.ds(L * tt, L)], sT[p]).wait()

        def wait_out(p):
            pltpu.make_async_copy(Ob[p].at[pl.ds(0, L)],
                                  out.at[b, pl.ds(0, L), :], sO[p]).wait()

        def compute_unit(g, p):
            i0 = unit_i0(g)
            q = i0 // L
            cslab = (P - 1) - i0 - SLABW * (g >> 3)
            T, O = Tb[p], Ob[p]

            def row_body(li):
                i = i0 + li
                ccv = (cslab - li) + iota * 0
                liv = li + iota * 0

                @plsc.parallel_loop(0, q + 1, unroll=8,
                                    carry=(PAD + i) - iota)
                def _(m, rv):
                    O[li, pl.ds(m * L, L)] = plsc.load_gather(S, [rv, ccv])
                    return rv - L

                rm = liv - iota
                gS = plsc.load_gather(S, [jnp.maximum(rm, 0), ccv])
                gT = plsc.load_gather(T, [liv, jnp.maximum(iota - li - 1, 0)])
                O[li, pl.ds((q + 1) * L, L)] = jnp.where(rm >= 0, gS, gT)

                cv0 = (L - 1 - li - (q + 2) * L) + iota

                @plsc.parallel_loop(q + 2, NCHUNK, unroll=8)
                def _(m):
                    cv = cv0 + m * L
                    rowT = ((cv >> 7) << 4) + liv
                    O[li, pl.ds(m * L, L)] = plsc.load_gather(
                        T, [rowT, cv & (SLABW - 1)])

            plsc.parallel_loop(0, L, unroll=2)(row_body)
            pltpu.async_copy(O.at[pl.ds(0, L)], out.at[b, pl.ds(i0, L), :],
                             sO[p])

        issue_in(0, 0)
        issue_in(1, 1)

        def pair_body(j, carry):
            issue_slab(j)
            for p in (0, 1):
                g = 2 * j + p
                wait_in(p)

                @pl.when(j > 0)
                def _():
                    wait_out(p)

                compute_unit(g, p)

                @pl.when(j < NU // 2 - 1)
                def _():
                    issue_in(g + 2, p)
            return carry

        lax.fori_loop(0, NU // 2, pair_body, 0)
        wait_out(0)
        wait_out(1)

    return shear_kernel


def kernel(triangle01, triangle02):
    if "k" not in _cached:
        _cached["k"] = _build()
    return _cached["k"](triangle01, triangle02)
